# baseline (speedup 1.0000x reference)
import functools

import jax
import jax.numpy as jnp
from jax import lax
from jax.experimental import pallas as pl
from jax.experimental.pallas import tpu as pltpu
from jax.experimental.pallas import tpu_sc as plsc

B, P, G, H, W = 4, 20000, 200, 512, 512
PPAD = 20480
BLK = 4096
NP = PPAD // BLK
NWORK = 32
CHUNK = (B * PPAD) // NWORK
NGATH = CHUNK // 128
VPB = CHUNK // 16
NPADTAIL = PPAD - P
MAGIC = 2.0 ** 23


def _sc_gather(xs, ys, masks_flat, tvx, tvy):
    mesh = plsc.VectorSubcoreMesh(core_axis_name="c", subcore_axis_name="s")
    n = B * PPAD

    @functools.partial(
        pl.kernel,
        mesh=mesh,
        compiler_params=pltpu.CompilerParams(needs_layout_passes=False),
        out_type=[
            jax.ShapeDtypeStruct((n,), jnp.int32),
            jax.ShapeDtypeStruct((n,), jnp.float32),
            jax.ShapeDtypeStruct((n,), jnp.float32),
        ],
        scratch_types=[
            pltpu.VMEM((CHUNK,), jnp.float32),
            pltpu.VMEM((CHUNK,), jnp.float32),
            pltpu.VMEM((CHUNK,), jnp.int32),
            pltpu.VMEM((CHUNK,), jnp.int32),
            pltpu.VMEM((G,), jnp.float32),
            pltpu.VMEM((G,), jnp.float32),
            pltpu.VMEM((CHUNK,), jnp.float32),
            pltpu.VMEM((CHUNK,), jnp.float32),
            pltpu.SemaphoreType.DMA,
        ],
    )
    def sc_kernel(xs_hbm, ys_hbm, masks_hbm, tvx_hbm, tvy_hbm,
                  lab_hbm, vxg_hbm, vyg_hbm,
                  xv, yv, idxv, labv, tvx_v, tvy_v, vxv, vyv, sem):
        nc = 2
        wid = lax.axis_index("s") * nc + lax.axis_index("c")
        base = wid * CHUNK
        batch = base // PPAD
        hoff = batch * (H * W)
        pltpu.sync_copy(xs_hbm.at[pl.ds(base, CHUNK)], xv)
        pltpu.sync_copy(ys_hbm.at[pl.ds(base, CHUNK)], yv)
        pltpu.sync_copy(tvx_hbm.at[pl.ds(batch * G, G)], tvx_v)
        pltpu.sync_copy(tvy_hbm.at[pl.ds(batch * G, G)], tvy_v)

        copies = []
        for c in range(NGATH):
            def body(i, carry, c=c):
                o = c * 128 + i * 16
                x16 = xv[pl.ds(o, 16)]
                y16 = yv[pl.ds(o, 16)]
                rx = (x16 + MAGIC) - MAGIC
                ry = (y16 + MAGIC) - MAGIC
                rx = jnp.minimum(jnp.maximum(rx, 0.0), float(W - 1))
                ry = jnp.minimum(jnp.maximum(ry, 0.0), float(H - 1))
                xi = rx.astype(jnp.int32)
                yi = ry.astype(jnp.int32)
                idxv[pl.ds(o, 16)] = yi * W + xi + hoff
                return carry

            lax.fori_loop(0, 8, body, 0)
            copies.append(
                pltpu.async_copy(
                    masks_hbm.at[idxv.at[pl.ds(c * 128, 128)]],
                    labv.at[pl.ds(c * 128, 128)],
                    sem,
                )
            )

        last_in_batch = wid % (PPAD // CHUNK) == (PPAD // CHUNK) - 1
        for c in range(NGATH):
            copies[c].wait()

            zlo = max(c * 128, CHUNK - NPADTAIL)
            zhi = (c + 1) * 128
            if zlo < zhi:
                @pl.when(last_in_batch)
                def _zero_pad(zlo=zlo, zhi=zhi):
                    def zbody(i, carry):
                        labv[pl.ds(zlo + i * 16, 16)] = jnp.zeros(
                            (16,), jnp.int32
                        )
                        return carry

                    lax.fori_loop(0, (zhi - zlo) // 16, zbody, 0)

            def tbody(i, carry, c=c):
                o = c * 128 + i * 16
                lab16 = labv[pl.ds(o, 16)]
                t16 = jnp.maximum(lab16 - 1, 0)
                vxv[pl.ds(o, 16)] = plsc.load_gather(tvx_v, [t16])
                vyv[pl.ds(o, 16)] = plsc.load_gather(tvy_v, [t16])
                return carry

            lax.fori_loop(0, 8, tbody, 0)

        pltpu.sync_copy(labv, lab_hbm.at[pl.ds(base, CHUNK)])
        pltpu.sync_copy(vxv, vxg_hbm.at[pl.ds(base, CHUNK)])
        pltpu.sync_copy(vyv, vyg_hbm.at[pl.ds(base, CHUNK)])

    return sc_kernel(xs, ys, masks_flat, tvx, tvy)


def _ieq(a, b):
    return lax.bitcast_convert_type(a, jnp.int32) == lax.bitcast_convert_type(
        b, jnp.int32
    )


def _tc_global_kernel(pts_ref, tgt_ref, gmin_ref, gidx_ref):
    ip = pl.program_id(1)
    inf = jnp.float32(jnp.inf)
    bigf = jnp.float32(1e9)

    @pl.when(ip == 0)
    def _init():
        gmin_ref[0, :, :] = jnp.full((G, 1), inf, jnp.float32)
        gidx_ref[0, :, :] = jnp.zeros((G, 1), jnp.float32)

    ux = pts_ref[0, 0:1, :]
    uy = pts_ref[0, 1:2, :]
    vx = tgt_ref[0, :, 0:1]
    vy = tgt_ref[0, :, 1:2]
    dx = ux - vx
    dy = uy - vy
    s = jnp.sqrt(dx * dx + dy * dy + jnp.float32(1e-12))

    pidf = jnp.float32(ip * BLK) + lax.broadcasted_iota(
        jnp.int32, (1, BLK), 1).astype(jnp.float32)
    pidb = jnp.broadcast_to(pidf, (G, BLK))

    bgmin = jnp.min(s, axis=1, keepdims=True)
    bgidx = jnp.min(jnp.where(_ieq(s, bgmin), pidb, bigf), axis=1,
                    keepdims=True)

    gm = gmin_ref[0, :, :]
    gidx_ref[0, :, :] = jnp.where(bgmin < gm, bgidx, gidx_ref[0, :, :])
    gmin_ref[0, :, :] = jnp.minimum(bgmin, gm)


def _tc_inside_kernel(pts_ref, lab_ref, vxg_ref, vyg_ref,
                      imin_ref, iidx_ref, src_ref, cost_ref):
    ip = pl.program_id(1)
    inf = jnp.float32(jnp.inf)
    bigf = jnp.float32(1e9)

    @pl.when(ip == 0)
    def _init():
        imin_ref[0, :, :] = jnp.full((G, 1), inf, jnp.float32)
        iidx_ref[0, :, :] = jnp.zeros((G, 1), jnp.float32)

    ux = pts_ref[0, 0:1, :]
    uy = pts_ref[0, 1:2, :]
    vx = vxg_ref[0, :, :]
    vy = vyg_ref[0, :, :]
    dx = ux - vx
    dy = uy - vy
    s_own = jnp.sqrt(dx * dx + dy * dy + jnp.float32(1e-12))

    lab = lab_ref[0, :, :]
    ids = lax.broadcasted_iota(jnp.int32, (G, 1), 0) + 1
    inside = lab == ids
    s_i = jnp.where(inside, jnp.broadcast_to(s_own, (G, BLK)), inf)

    pidf = jnp.float32(ip * BLK) + lax.broadcasted_iota(
        jnp.int32, (1, BLK), 1).astype(jnp.float32)
    pidb = jnp.broadcast_to(pidf, (G, BLK))

    bimin = jnp.min(s_i, axis=1, keepdims=True)
    biidx = jnp.min(jnp.where(_ieq(s_i, bimin), pidb, bigf), axis=1,
                    keepdims=True)

    im = imin_ref[0, :, :]
    iidx_ref[0, :, :] = jnp.where(bimin < im, biidx, iidx_ref[0, :, :])
    imin_ref[0, :, :] = jnp.minimum(bimin, im)

    @pl.when(ip == NP - 1)
    def _fin():
        src_ref[0, :, :] = iidx_ref[0, :, :].astype(jnp.int32)
        cost_ref[0, :, :] = jnp.sum(imin_ref[0, :, :], axis=0, keepdims=True)


def _tc_global(pts_t, tgt, interpret=False):
    return pl.pallas_call(
        _tc_global_kernel,
        grid=(B, NP),
        in_specs=[
            pl.BlockSpec((1, 2, BLK), lambda b, i: (b, 0, i)),
            pl.BlockSpec((1, G, 2), lambda b, i: (b, 0, 0)),
        ],
        out_specs=[
            pl.BlockSpec((1, G, 1), lambda b, i: (b, 0, 0)),
            pl.BlockSpec((1, G, 1), lambda b, i: (b, 0, 0)),
        ],
        out_shape=[
            jax.ShapeDtypeStruct((B, G, 1), jnp.float32),
            jax.ShapeDtypeStruct((B, G, 1), jnp.float32),
        ],
        interpret=interpret,
    )(pts_t, tgt)


def _tc_inside(pts_t, labels3, vxg3, vyg3, interpret=False):
    return pl.pallas_call(
        _tc_inside_kernel,
        grid=(B, NP),
        in_specs=[
            pl.BlockSpec((1, 2, BLK), lambda b, i: (b, 0, i)),
            pl.BlockSpec((1, 1, BLK), lambda b, i: (b, 0, i)),
            pl.BlockSpec((1, 1, BLK), lambda b, i: (b, 0, i)),
            pl.BlockSpec((1, 1, BLK), lambda b, i: (b, 0, i)),
        ],
        out_specs=[
            pl.BlockSpec((1, G, 1), lambda b, i: (b, 0, 0)),
            pl.BlockSpec((1, G, 1), lambda b, i: (b, 0, 0)),
            pl.BlockSpec((1, G, 1), lambda b, i: (b, 0, 0)),
            pl.BlockSpec((1, 1, 1), lambda b, i: (b, 0, 0)),
        ],
        out_shape=[
            jax.ShapeDtypeStruct((B, G, 1), jnp.float32),
            jax.ShapeDtypeStruct((B, G, 1), jnp.float32),
            jax.ShapeDtypeStruct((B, G, 1), jnp.int32),
            jax.ShapeDtypeStruct((B, 1, 1), jnp.float32),
        ],
        interpret=interpret,
    )(pts_t, labels3, vxg3, vyg3)


def kernel(pred_points, target_points, target_masks):
    pad = PPAD - P
    xs = jnp.pad(pred_points[:, :, 0], ((0, 0), (0, pad))).reshape(-1)
    ys = jnp.pad(pred_points[:, :, 1], ((0, 0), (0, pad))).reshape(-1)
    masks_flat = target_masks.reshape(-1)
    tvx = target_points[:, :, 0].reshape(-1)
    tvy = target_points[:, :, 1].reshape(-1)

    labels, vxg, vyg = _sc_gather(xs, ys, masks_flat, tvx, tvy)
    labels3 = labels.reshape(B, 1, PPAD)
    vxg3 = vxg.reshape(B, 1, PPAD)
    vyg3 = vyg.reshape(B, 1, PPAD)

    pts_t = jnp.pad(
        jnp.swapaxes(pred_points, 1, 2),
        ((0, 0), (0, 0), (0, pad)),
        constant_values=1e6,
    )
    imin3, iidx3, src3, cost3 = _tc_inside(pts_t, labels3, vxg3, vyg3)
    costs_i = cost3[:, 0, 0]

    def _with_global(_):
        gmin3, gidx3 = _tc_global(pts_t, target_points)
        imin = imin3[:, :, 0]
        iidx = iidx3[:, :, 0]
        has = ~jnp.isinf(imin)
        sel_min = jnp.where(has, imin, gmin3[:, :, 0])
        sel_idx = jnp.where(has, iidx, gidx3[:, :, 0])
        return sel_idx.astype(jnp.int32), jnp.sum(sel_min, axis=1)

    def _inside_only(_):
        return src3[:, :, 0], costs_i

    src, costs = lax.cond(
        jnp.all(jnp.isfinite(costs_i)), _inside_only, _with_global, None
    )
    tgt = jnp.broadcast_to(jnp.arange(G, dtype=jnp.int32), (B, G))
    return src, tgt, costs


# speedup vs baseline: 1.0470x; 1.0470x over previous
import functools

import jax
import jax.numpy as jnp
from jax import lax
from jax.experimental import pallas as pl
from jax.experimental.pallas import tpu as pltpu
from jax.experimental.pallas import tpu_sc as plsc

B, P, G, H, W = 4, 20000, 200, 512, 512
PPAD = 20480
BLK = 10240
NP = PPAD // BLK
NWORK = 32
CHUNK = (B * PPAD) // NWORK
NGATH = CHUNK // 128
VPB = CHUNK // 16
NPADTAIL = PPAD - P
MAGIC = 2.0 ** 23


def _sc_gather(xs, ys, masks_flat, tvx, tvy):
    mesh = plsc.VectorSubcoreMesh(core_axis_name="c", subcore_axis_name="s")
    n = B * PPAD

    @functools.partial(
        pl.kernel,
        mesh=mesh,
        compiler_params=pltpu.CompilerParams(needs_layout_passes=False),
        out_type=[
            jax.ShapeDtypeStruct((n,), jnp.int32),
            jax.ShapeDtypeStruct((n,), jnp.float32),
            jax.ShapeDtypeStruct((n,), jnp.float32),
        ],
        scratch_types=[
            pltpu.VMEM((CHUNK,), jnp.float32),
            pltpu.VMEM((CHUNK,), jnp.float32),
            pltpu.VMEM((CHUNK,), jnp.int32),
            pltpu.VMEM((CHUNK,), jnp.int32),
            pltpu.VMEM((G,), jnp.float32),
            pltpu.VMEM((G,), jnp.float32),
            pltpu.VMEM((CHUNK,), jnp.float32),
            pltpu.VMEM((CHUNK,), jnp.float32),
            pltpu.SemaphoreType.DMA,
        ],
    )
    def sc_kernel(xs_hbm, ys_hbm, masks_hbm, tvx_hbm, tvy_hbm,
                  lab_hbm, vxg_hbm, vyg_hbm,
                  xv, yv, idxv, labv, tvx_v, tvy_v, vxv, vyv, sem):
        nc = 2
        wid = lax.axis_index("s") * nc + lax.axis_index("c")
        base = wid * CHUNK
        batch = base // PPAD
        hoff = batch * (H * W)
        pltpu.sync_copy(xs_hbm.at[pl.ds(base, CHUNK)], xv)
        pltpu.sync_copy(ys_hbm.at[pl.ds(base, CHUNK)], yv)
        pltpu.sync_copy(tvx_hbm.at[pl.ds(batch * G, G)], tvx_v)
        pltpu.sync_copy(tvy_hbm.at[pl.ds(batch * G, G)], tvy_v)

        def body(i, carry):
            x16 = xv[pl.ds(i * 16, 16)]
            y16 = yv[pl.ds(i * 16, 16)]
            rx = (x16 + MAGIC) - MAGIC
            ry = (y16 + MAGIC) - MAGIC
            rx = jnp.minimum(jnp.maximum(rx, 0.0), float(W - 1))
            ry = jnp.minimum(jnp.maximum(ry, 0.0), float(H - 1))
            xi = rx.astype(jnp.int32)
            yi = ry.astype(jnp.int32)
            idxv[pl.ds(i * 16, 16)] = yi * W + xi + hoff
            return carry

        lax.fori_loop(0, VPB, body, 0)

        copies = []
        for c in range(NGATH):
            copies.append(
                pltpu.async_copy(
                    masks_hbm.at[idxv.at[pl.ds(c * 128, 128)]],
                    labv.at[pl.ds(c * 128, 128)],
                    sem,
                )
            )
        for cp in copies:
            cp.wait()

        @pl.when(wid % (PPAD // CHUNK) == (PPAD // CHUNK) - 1)
        def _zero_pad():
            def zbody(i, carry):
                labv[pl.ds((CHUNK - NPADTAIL) + i * 16, 16)] = jnp.zeros(
                    (16,), jnp.int32
                )
                return carry

            lax.fori_loop(0, NPADTAIL // 16, zbody, 0)

        def tbody(i, carry):
            lab16 = labv[pl.ds(i * 16, 16)]
            t16 = jnp.maximum(lab16 - 1, 0)
            vxv[pl.ds(i * 16, 16)] = plsc.load_gather(tvx_v, [t16])
            vyv[pl.ds(i * 16, 16)] = plsc.load_gather(tvy_v, [t16])
            return carry

        lax.fori_loop(0, VPB, tbody, 0)

        pltpu.sync_copy(labv, lab_hbm.at[pl.ds(base, CHUNK)])
        pltpu.sync_copy(vxv, vxg_hbm.at[pl.ds(base, CHUNK)])
        pltpu.sync_copy(vyv, vyg_hbm.at[pl.ds(base, CHUNK)])

    return sc_kernel(xs, ys, masks_flat, tvx, tvy)


def _ieq(a, b):
    return lax.bitcast_convert_type(a, jnp.int32) == lax.bitcast_convert_type(
        b, jnp.int32
    )


def _tc_global_kernel(pts_ref, tgt_ref, gmin_ref, gidx_ref):
    ip = pl.program_id(1)
    inf = jnp.float32(jnp.inf)
    bigf = jnp.float32(1e9)

    @pl.when(ip == 0)
    def _init():
        gmin_ref[0, :, :] = jnp.full((G, 1), inf, jnp.float32)
        gidx_ref[0, :, :] = jnp.zeros((G, 1), jnp.float32)

    ux = pts_ref[0, 0:1, :]
    uy = pts_ref[0, 1:2, :]
    vx = tgt_ref[0, :, 0:1]
    vy = tgt_ref[0, :, 1:2]
    dx = ux - vx
    dy = uy - vy
    s = jnp.sqrt(dx * dx + dy * dy + jnp.float32(1e-12))

    pidf = jnp.float32(ip * BLK) + lax.broadcasted_iota(
        jnp.int32, (1, BLK), 1).astype(jnp.float32)
    pidb = jnp.broadcast_to(pidf, (G, BLK))

    bgmin = jnp.min(s, axis=1, keepdims=True)
    bgidx = jnp.min(jnp.where(_ieq(s, bgmin), pidb, bigf), axis=1,
                    keepdims=True)

    gm = gmin_ref[0, :, :]
    gidx_ref[0, :, :] = jnp.where(bgmin < gm, bgidx, gidx_ref[0, :, :])
    gmin_ref[0, :, :] = jnp.minimum(bgmin, gm)


def _tc_inside_kernel(pts_ref, lab_ref, vxg_ref, vyg_ref,
                      imin_ref, iidx_ref, src_ref, cost_ref):
    ip = pl.program_id(1)
    inf = jnp.float32(jnp.inf)
    bigf = jnp.float32(1e9)

    @pl.when(ip == 0)
    def _init():
        imin_ref[0, :, :] = jnp.full((G, 1), inf, jnp.float32)
        iidx_ref[0, :, :] = jnp.zeros((G, 1), jnp.float32)

    ux = pts_ref[0, 0:1, :]
    uy = pts_ref[0, 1:2, :]
    vx = vxg_ref[0, :, :]
    vy = vyg_ref[0, :, :]
    dx = ux - vx
    dy = uy - vy
    s_own = jnp.sqrt(dx * dx + dy * dy + jnp.float32(1e-12))

    lab = lab_ref[0, :, :]
    ids = lax.broadcasted_iota(jnp.int32, (G, 1), 0) + 1
    inside = lab == ids
    s_i = jnp.where(inside, jnp.broadcast_to(s_own, (G, BLK)), inf)

    pidf = jnp.float32(ip * BLK) + lax.broadcasted_iota(
        jnp.int32, (1, BLK), 1).astype(jnp.float32)
    pidb = jnp.broadcast_to(pidf, (G, BLK))

    bimin = jnp.min(s_i, axis=1, keepdims=True)
    biidx = jnp.min(jnp.where(_ieq(s_i, bimin), pidb, bigf), axis=1,
                    keepdims=True)

    im = imin_ref[0, :, :]
    iidx_ref[0, :, :] = jnp.where(bimin < im, biidx, iidx_ref[0, :, :])
    imin_ref[0, :, :] = jnp.minimum(bimin, im)

    @pl.when(ip == NP - 1)
    def _fin():
        src_ref[0, :, :] = iidx_ref[0, :, :].astype(jnp.int32)
        cost_ref[0, :, :] = jnp.sum(imin_ref[0, :, :], axis=0, keepdims=True)


def _tc_global(pts_t, tgt, interpret=False):
    return pl.pallas_call(
        _tc_global_kernel,
        grid=(B, NP),
        in_specs=[
            pl.BlockSpec((1, 2, BLK), lambda b, i: (b, 0, i)),
            pl.BlockSpec((1, G, 2), lambda b, i: (b, 0, 0)),
        ],
        out_specs=[
            pl.BlockSpec((1, G, 1), lambda b, i: (b, 0, 0)),
            pl.BlockSpec((1, G, 1), lambda b, i: (b, 0, 0)),
        ],
        out_shape=[
            jax.ShapeDtypeStruct((B, G, 1), jnp.float32),
            jax.ShapeDtypeStruct((B, G, 1), jnp.float32),
        ],
        interpret=interpret,
    )(pts_t, tgt)


def _tc_inside(pts_t, labels3, vxg3, vyg3, interpret=False):
    return pl.pallas_call(
        _tc_inside_kernel,
        grid=(B, NP),
        in_specs=[
            pl.BlockSpec((1, 2, BLK), lambda b, i: (b, 0, i)),
            pl.BlockSpec((1, 1, BLK), lambda b, i: (b, 0, i)),
            pl.BlockSpec((1, 1, BLK), lambda b, i: (b, 0, i)),
            pl.BlockSpec((1, 1, BLK), lambda b, i: (b, 0, i)),
        ],
        out_specs=[
            pl.BlockSpec((1, G, 1), lambda b, i: (b, 0, 0)),
            pl.BlockSpec((1, G, 1), lambda b, i: (b, 0, 0)),
            pl.BlockSpec((1, G, 1), lambda b, i: (b, 0, 0)),
            pl.BlockSpec((1, 1, 1), lambda b, i: (b, 0, 0)),
        ],
        out_shape=[
            jax.ShapeDtypeStruct((B, G, 1), jnp.float32),
            jax.ShapeDtypeStruct((B, G, 1), jnp.float32),
            jax.ShapeDtypeStruct((B, G, 1), jnp.int32),
            jax.ShapeDtypeStruct((B, 1, 1), jnp.float32),
        ],
        interpret=interpret,
    )(pts_t, labels3, vxg3, vyg3)


def kernel(pred_points, target_points, target_masks):
    pad = PPAD - P
    xs = jnp.pad(pred_points[:, :, 0], ((0, 0), (0, pad))).reshape(-1)
    ys = jnp.pad(pred_points[:, :, 1], ((0, 0), (0, pad))).reshape(-1)
    masks_flat = target_masks.reshape(-1)
    tvx = target_points[:, :, 0].reshape(-1)
    tvy = target_points[:, :, 1].reshape(-1)

    labels, vxg, vyg = _sc_gather(xs, ys, masks_flat, tvx, tvy)
    labels3 = labels.reshape(B, 1, PPAD)
    vxg3 = vxg.reshape(B, 1, PPAD)
    vyg3 = vyg.reshape(B, 1, PPAD)

    pts_t = jnp.pad(
        jnp.swapaxes(pred_points, 1, 2),
        ((0, 0), (0, 0), (0, pad)),
        constant_values=1e6,
    )
    imin3, iidx3, src3, cost3 = _tc_inside(pts_t, labels3, vxg3, vyg3)
    costs_i = cost3[:, 0, 0]

    def _with_global(_):
        gmin3, gidx3 = _tc_global(pts_t, target_points)
        imin = imin3[:, :, 0]
        iidx = iidx3[:, :, 0]
        has = ~jnp.isinf(imin)
        sel_min = jnp.where(has, imin, gmin3[:, :, 0])
        sel_idx = jnp.where(has, iidx, gidx3[:, :, 0])
        return sel_idx.astype(jnp.int32), jnp.sum(sel_min, axis=1)

    def _inside_only(_):
        return src3[:, :, 0], costs_i

    src, costs = lax.cond(
        jnp.all(jnp.isfinite(costs_i)), _inside_only, _with_global, None
    )
    tgt = jnp.broadcast_to(jnp.arange(G, dtype=jnp.int32), (B, G))
    return src, tgt, costs
